# baseline (speedup 1.0000x reference)
import functools

import jax
import jax.numpy as jnp
from jax import lax
from jax.experimental import pallas as pl
from jax.experimental.pallas import tpu as pltpu
from jax.experimental.pallas import tpu_sc as plsc

_CHUNK = 128


def kernel(x, idx):
    b, t, _ = x.shape
    num = idx.shape[1]
    info = plsc.get_sparse_core_info()
    lanes = info.num_lanes
    nc, ns = info.num_cores, info.num_subcores
    nw = nc * ns
    rows_per_w = b // nw
    nchunk = num // _CHUNK
    nbuf = 3

    mesh = plsc.VectorSubcoreMesh(core_axis_name="c", subcore_axis_name="s")

    @functools.partial(
        pl.kernel,
        out_type=jax.ShapeDtypeStruct((b * t,), jnp.float32),
        mesh=mesh,
        compiler_params=pltpu.CompilerParams(needs_layout_passes=False),
        scratch_types=[
            pltpu.VMEM_SHARED((ns * nbuf * t,), jnp.float32),
            pltpu.VMEM((rows_per_w, num), jnp.int32),
            pltpu.VMEM((2, nchunk, _CHUNK), jnp.int32),
            pltpu.VMEM((_CHUNK,), jnp.float32),
            pltpu.SemaphoreType.DMA,
            pltpu.SemaphoreType.DMA,
            pltpu.SemaphoreType.DMA,
            pltpu.SemaphoreType.DMA,
            pltpu.SemaphoreType.DMA,
            pltpu.SemaphoreType.DMA,
            pltpu.SemaphoreType.DMA,
            pltpu.SemaphoreType.DMA,
        ],
    )
    def cut(x_hbm, idx_hbm, out_hbm, sbuf, idx_v, stage, zbuf,
            semi, semsc, semx0, semx1, semx2, semo0, semo1, semo2):
        cid = lax.axis_index("c")
        sid = lax.axis_index("s")
        wid = sid * nc + cid
        base = wid * rows_per_w
        semx = (semx0, semx1, semx2)
        semo = (semo0, semo1, semo2)

        z16 = jnp.zeros((lanes,), jnp.float32)
        for m in range(_CHUNK // lanes):
            zbuf[pl.ds(m * lanes, lanes)] = z16

        ci = pltpu.async_copy(idx_hbm.at[pl.ds(base, rows_per_w)], idx_v, semi)
        loads = [None] * rows_per_w
        stores = [None] * rows_per_w

        def buf_off(i):
            return (sid * nbuf + i % nbuf) * t

        def start_load(i):
            loads[i] = pltpu.async_copy(
                x_hbm.at[pl.ds((base + i) * t, t)],
                sbuf.at[pl.ds(buf_off(i), t)], semx[i % nbuf])

        start_load(0)
        start_load(1)
        ci.wait()
        for i in range(rows_per_w):
            if i + 2 < rows_per_w:
                if i >= 1:
                    stores[i - 1].wait()
                start_load(i + 2)
            p = i % 2
            off = buf_off(i)

            @plsc.parallel_loop(0, nchunk, carry=None)
            def _(c, i=i, p=p, off=off):
                for m in range(_CHUNK // lanes):
                    v = idx_v[i, pl.ds(c * _CHUNK + m * lanes, lanes)] + off
                    stage[p, c, pl.ds(m * lanes, lanes)] = v

            loads[i].wait()
            scs = [
                pltpu.async_copy(zbuf, sbuf.at[stage.at[p].at[c]], semsc)
                for c in range(nchunk)
            ]
            for s in scs:
                s.wait()
            stores[i] = pltpu.async_copy(
                sbuf.at[pl.ds(off, t)],
                out_hbm.at[pl.ds((base + i) * t, t)], semo[i % nbuf])
        for s in stores[-nbuf:]:
            s.wait()

    out = cut(jnp.reshape(x, (b * t,)), idx)
    return jnp.reshape(out, (b, t, 1))


# speedup vs baseline: 1.1194x; 1.1194x over previous
import functools

import jax
import jax.numpy as jnp
from jax import lax
from jax.experimental import pallas as pl
from jax.experimental.pallas import tpu as pltpu
from jax.experimental.pallas import tpu_sc as plsc


def kernel(x, idx):
    b, t, _ = x.shape
    num = idx.shape[1]
    info = plsc.get_sparse_core_info()
    lanes = info.num_lanes
    nw = info.num_cores * info.num_subcores
    rows_per_w = b // nw

    mesh = plsc.VectorSubcoreMesh(core_axis_name="c", subcore_axis_name="s")

    @functools.partial(
        pl.kernel,
        out_type=jax.ShapeDtypeStruct((b * t,), jnp.float32),
        mesh=mesh,
        compiler_params=pltpu.CompilerParams(needs_layout_passes=False),
        scratch_types=[
            pltpu.VMEM((t,), jnp.float32),
            pltpu.VMEM((t,), jnp.float32),
            pltpu.VMEM((t,), jnp.float32),
            pltpu.VMEM((rows_per_w, num), jnp.int32),
            pltpu.SemaphoreType.DMA,
            pltpu.SemaphoreType.DMA,
            pltpu.SemaphoreType.DMA,
            pltpu.SemaphoreType.DMA,
            pltpu.SemaphoreType.DMA,
            pltpu.SemaphoreType.DMA,
            pltpu.SemaphoreType.DMA,
        ],
    )
    def cut(x_hbm, idx_hbm, out_hbm, row_v0, row_v1, row_v2, idx_v,
            semi, semx0, semx1, semx2, semo0, semo1, semo2):
        wid = lax.axis_index("s") * info.num_cores + lax.axis_index("c")
        base = wid * rows_per_w
        zeros = jnp.zeros((lanes,), jnp.float32)
        bufs = (row_v0, row_v1, row_v2)
        semx = (semx0, semx1, semx2)
        semo = (semo0, semo1, semo2)
        nbuf = 3

        ci = pltpu.async_copy(idx_hbm.at[pl.ds(base, rows_per_w)], idx_v, semi)
        loads = [None] * rows_per_w
        stores = [None] * rows_per_w

        def start_load(i):
            loads[i] = pltpu.async_copy(
                x_hbm.at[pl.ds((base + i) * t, t)], bufs[i % nbuf], semx[i % nbuf])

        start_load(0)
        start_load(1)
        ci.wait()
        for i in range(rows_per_w):
            if i + 2 < rows_per_w:
                if i >= 1:
                    stores[i - 1].wait()
                start_load(i + 2)
            loads[i].wait()

            @plsc.parallel_loop(0, num, step=lanes, unroll=8)
            def _(j, i=i):
                v = idx_v[i, pl.ds(j, lanes)]
                plsc.store_scatter(bufs[i % nbuf], [v], zeros)
            stores[i] = pltpu.async_copy(
                bufs[i % nbuf], out_hbm.at[pl.ds((base + i) * t, t)], semo[i % nbuf])
        for s in stores[-3:]:
            s.wait()

    out = cut(jnp.reshape(x, (b * t,)), idx)
    return jnp.reshape(out, (b, t, 1))
